# Initial kernel scaffold; baseline (speedup 1.0000x reference)
import jax
import jax.numpy as jnp
from jax.experimental import pallas as pl


def kernel(x, edge_attr, edge_index, W_ne, b_ne, W_ee, b_ee, W_e0, b_e0, W_n0, b_n0, W_e1, b_e1, W_n1, b_n1, W_cls, b_cls):
    raise NotImplementedError("write your pallas kernel here")


# speedup vs baseline: 1.7056x; 1.7056x over previous
import functools

import jax
import jax.numpy as jnp
import numpy as np
from jax import lax
from jax.experimental import pallas as pl
from jax.experimental.pallas import tpu as pltpu
from jax.experimental.pallas import tpu_sc as plsc

N_NODES = 10000
N_EDGES = 320000
D = 128

BN_SCALE = float(1.0 / np.sqrt(1.0 + 1e-5))


_NC = 2
_NS = 16
_NW = _NC * _NS
_EPW = N_EDGES // _NW
_CHUNK = 80
_NCHUNK = _EPW // _CHUNK

_mesh = plsc.VectorSubcoreMesh(core_axis_name="c", subcore_axis_name="s")


def _gather_body(with_c, *refs):
    if with_c:
        (ta, tb, tc_, ii_hbm, jj_hbm, g_out, ci_out,
         ii_v, jj_v, a_v, b_v, c_v, sem_a, sem_b, sem_c) = refs
    else:
        (ta, tb, ii_hbm, jj_hbm, g_out,
         ii_v, jj_v, a_v, b_v, sem_a, sem_b) = refs
    wid = lax.axis_index("s") * _NC + lax.axis_index("c")
    base = wid * _EPW

    def chunk(ch, carry):
        off = base + ch * _CHUNK
        pltpu.sync_copy(ii_hbm.at[pl.ds(off, _CHUNK)], ii_v)
        pltpu.sync_copy(jj_hbm.at[pl.ds(off, _CHUNK)], jj_v)
        cp_a = pltpu.async_copy(ta.at[ii_v], a_v, sem_a)
        cp_b = pltpu.async_copy(tb.at[jj_v], b_v, sem_b)
        if with_c:
            cp_c = pltpu.async_copy(tc_.at[ii_v], c_v, sem_c)
        cp_a.wait()
        cp_b.wait()

        def row(r, rc):
            for q in range(D // 16):
                sl = pl.ds(q * 16, 16)
                a_v[r, sl] = a_v[r, sl] + b_v[r, sl]
            return rc

        lax.fori_loop(0, _CHUNK, row, 0, unroll=2)
        pltpu.sync_copy(a_v, g_out.at[pl.ds(off, _CHUNK)])
        if with_c:
            cp_c.wait()
            pltpu.sync_copy(c_v, ci_out.at[pl.ds(off, _CHUNK)])
        return carry

    lax.fori_loop(0, _NCHUNK, chunk, 0)


def _make_gather(with_c):
    out_type = [jax.ShapeDtypeStruct((N_EDGES, D), jnp.float32)]
    scratch = [
        pltpu.VMEM((_CHUNK,), jnp.int32),
        pltpu.VMEM((_CHUNK,), jnp.int32),
        pltpu.VMEM((_CHUNK, D), jnp.float32),
        pltpu.VMEM((_CHUNK, D), jnp.float32),
    ]
    if with_c:
        out_type = out_type + [jax.ShapeDtypeStruct((N_EDGES, D), jnp.float32)]
        scratch = scratch + [pltpu.VMEM((_CHUNK, D), jnp.float32)]
    scratch = scratch + [pltpu.SemaphoreType.DMA] * (3 if with_c else 2)
    return pl.kernel(
        functools.partial(_gather_body, with_c),
        mesh=_mesh,
        out_type=out_type,
        scratch_types=scratch,
    )


_gather_ac = _make_gather(True)
_gather_a = _make_gather(False)

_NPAD = 10240
_ROWS_PER_TILE = _NPAD // _NS
_ZROWS = 128


def _scatter_body(msg_hbm, ii_hbm, out_hbm, acc_sh, m_v, ii_v, z_v, sem):
    cid = lax.axis_index("c")
    sid = lax.axis_index("s")

    zv = jnp.zeros((16,), jnp.float32)

    def zrow(r, rc):
        for q in range(D // 16):
            z_v[r, pl.ds(q * 16, 16)] = zv
        return rc

    lax.fori_loop(0, _ZROWS, zrow, 0)
    for q in range(_ROWS_PER_TILE // _ZROWS):
        pltpu.sync_copy(
            z_v, acc_sh.at[pl.ds(sid * _ROWS_PER_TILE + q * _ZROWS, _ZROWS)])
    plsc.subcore_barrier()

    base = (cid * _NS + sid) * _EPW

    def chunk(ch, carry):
        off = base + ch * _CHUNK
        pltpu.sync_copy(msg_hbm.at[pl.ds(off, _CHUNK)], m_v)
        pltpu.sync_copy(ii_hbm.at[pl.ds(off, _CHUNK)], ii_v)
        pltpu.sync_copy(m_v, acc_sh.at[ii_v], add=True)
        return carry

    lax.fori_loop(0, _NCHUNK, chunk, 0)
    plsc.subcore_barrier()

    for q in range(_ROWS_PER_TILE // _ZROWS):
        rows = sid * _ROWS_PER_TILE + q * _ZROWS
        pltpu.sync_copy(acc_sh.at[pl.ds(rows, _ZROWS)], z_v)
        pltpu.sync_copy(z_v, out_hbm.at[cid, pl.ds(rows, _ZROWS)])


_scatter_add = pl.kernel(
    _scatter_body,
    mesh=_mesh,
    out_type=jax.ShapeDtypeStruct((_NC, _NPAD, D), jnp.float32),
    scratch_types=[
        pltpu.VMEM_SHARED((_NPAD, D), jnp.float32),
        pltpu.VMEM((_CHUNK, D), jnp.float32),
        pltpu.VMEM((_CHUNK,), jnp.int32),
        pltpu.VMEM((_ZROWS, D), jnp.float32),
        pltpu.SemaphoreType.DMA,
    ],
)


def _np0_body(x_ref, wne, bne, wa, wb, wc, ta, tb, tc_):
    nf = jnp.maximum(
        jnp.dot(x_ref[...], wne[...], preferred_element_type=jnp.float32)
        + bne[...], 0.0)
    ta[...] = jnp.dot(nf, wa[...], preferred_element_type=jnp.float32)
    tb[...] = jnp.dot(nf, wb[...], preferred_element_type=jnp.float32)
    tc_[...] = jnp.dot(nf, wc[...], preferred_element_type=jnp.float32)


def _np1_body(p_ref, wa, wb, ta, tb):
    nf = p_ref[0] + p_ref[1]
    ta[...] = jnp.dot(nf, wa[...], preferred_element_type=jnp.float32)
    tb[...] = jnp.dot(nf, wb[...], preferred_element_type=jnp.float32)


_BE = 2560
_EGRID = N_EDGES // _BE


def _e0_body(g, ci, ea, wee, bee, we3, be_, wn2, bn_, wcls, bcls,
             t_out, msg_out, pred_out):
    ef0 = jnp.maximum(
        jnp.dot(ea[...], wee[...], preferred_element_type=jnp.float32)
        + bee[...], 0.0)
    t = BN_SCALE * jnp.maximum(
        g[...] + jnp.dot(ef0, we3[...], preferred_element_type=jnp.float32)
        + be_[...], 0.0)
    t_out[...] = t
    pred_out[...] = jnp.dot(t, wcls[...],
                            preferred_element_type=jnp.float32) + bcls[...]
    msg_out[...] = BN_SCALE * jnp.maximum(
        ci[...] + jnp.dot(t, wn2[...], preferred_element_type=jnp.float32)
        + bn_[...], 0.0)


def _e1_body(g, t0, we3, be_, wcls, bcls, pred_out):
    t = BN_SCALE * jnp.maximum(
        g[...] + jnp.dot(t0[...], we3[...], preferred_element_type=jnp.float32)
        + be_[...], 0.0)
    pred_out[...] = jnp.dot(t, wcls[...],
                            preferred_element_type=jnp.float32) + bcls[...]


def _full(shape):
    nd = len(shape)
    return pl.BlockSpec(shape, lambda i: (0,) * nd)


def _eblk(width):
    return pl.BlockSpec((_BE, width), lambda i: (i, 0))


def kernel(x, edge_attr, edge_index, W_ne, b_ne, W_ee, b_ee, W_e0, b_e0,
           W_n0, b_n0, W_e1, b_e1, W_n1, b_n1, W_cls, b_cls):
    f32 = jnp.float32
    jj = edge_index[0].astype(jnp.int32)
    ii = edge_index[1].astype(jnp.int32)

    wa0, wb0, we3_0 = W_e0[:D], W_e0[D:2 * D], W_e0[2 * D:]
    wc0, wn2_0 = W_n0[:D], W_n0[D:]
    wa1, wb1, we3_1 = W_e1[:D], W_e1[D:2 * D], W_e1[2 * D:]
    b_ne2 = b_ne.reshape(1, D)
    b_ee2 = b_ee.reshape(1, D)
    b_e02 = b_e0.reshape(1, D)
    b_n02 = b_n0.reshape(1, D)
    b_e12 = b_e1.reshape(1, D)
    b_cls2 = b_cls.reshape(1, 1)

    ta0, tb0, tc0 = pl.pallas_call(
        _np0_body,
        out_shape=[jax.ShapeDtypeStruct((N_NODES, D), f32)] * 3,
    )(x, W_ne, b_ne2, wa0, wb0, wc0)

    g0, ci0 = _gather_ac(ta0, tb0, tc0, ii, jj)

    t0, msg, pred0 = pl.pallas_call(
        _e0_body,
        grid=(_EGRID,),
        in_specs=[
            _eblk(D), _eblk(D), _eblk(16),
            _full((16, D)), _full((1, D)), _full((D, D)), _full((1, D)),
            _full((D, D)), _full((1, D)), _full((D, 1)), _full((1, 1)),
        ],
        out_specs=[_eblk(D), _eblk(D), _eblk(1)],
        out_shape=[
            jax.ShapeDtypeStruct((N_EDGES, D), f32),
            jax.ShapeDtypeStruct((N_EDGES, D), f32),
            jax.ShapeDtypeStruct((N_EDGES, 1), f32),
        ],
    )(g0, ci0, edge_attr, W_ee, b_ee2, we3_0, b_e02, wn2_0, b_n02,
      W_cls, b_cls2)

    partials = _scatter_add(msg, ii)[:, :N_NODES]

    ta1, tb1 = pl.pallas_call(
        _np1_body,
        out_shape=[jax.ShapeDtypeStruct((N_NODES, D), f32)] * 2,
    )(partials, wa1, wb1)

    (g1,) = _gather_a(ta1, tb1, ii, jj)

    pred1 = pl.pallas_call(
        _e1_body,
        grid=(_EGRID,),
        in_specs=[
            _eblk(D), _eblk(D),
            _full((D, D)), _full((1, D)), _full((D, 1)), _full((1, 1)),
        ],
        out_specs=_eblk(1),
        out_shape=jax.ShapeDtypeStruct((N_EDGES, 1), f32),
    )(g1, t0, we3_1, b_e12, W_cls, b_cls2)

    return (pred0.reshape(N_EDGES), pred1.reshape(N_EDGES))


# speedup vs baseline: 3.1458x; 1.8444x over previous
import functools

import jax
import jax.numpy as jnp
import numpy as np
from jax import lax
from jax.experimental import pallas as pl
from jax.experimental.pallas import tpu as pltpu
from jax.experimental.pallas import tpu_sc as plsc

N_NODES = 10000
N_EDGES = 320000
D = 128

BN_SCALE = float(1.0 / np.sqrt(1.0 + 1e-5))


_NC = 2
_NS = 16
_NW = _NC * _NS
_EPW = N_EDGES // _NW
_CHUNK = 80
_NCHUNK = _EPW // _CHUNK

_mesh = plsc.VectorSubcoreMesh(core_axis_name="c", subcore_axis_name="s")


def _gather_body(with_c, *refs):
    if with_c:
        (ta, tb, tc_, ii2, jj2, g_out, ci_out,
         ii_v, jj_v,
         a_v0, a_v1, b_v0, b_v1, c_v0, c_v1, o_v0, o_v1,
         sem_g0, sem_g1, sem_sg0, sem_sg1, sem_sc0, sem_sc1) = refs
        c_v = (c_v0, c_v1)
        sem_sc = (sem_sc0, sem_sc1)
    else:
        (ta, tb, ii2, jj2, g_out,
         ii_v, jj_v,
         a_v0, a_v1, b_v0, b_v1, o_v0, o_v1,
         sem_g0, sem_g1, sem_sg0, sem_sg1) = refs
    a_v = (a_v0, a_v1)
    b_v = (b_v0, b_v1)
    o_v = (o_v0, o_v1)
    sem_g = (sem_g0, sem_g1)
    sem_sg = (sem_sg0, sem_sg1)

    wid = lax.axis_index("s") * _NC + lax.axis_index("c")
    ebase = wid * _EPW

    pltpu.sync_copy(ii2.at[wid], ii_v)
    pltpu.sync_copy(jj2.at[wid], jj_v)

    def start(lc, b, first):
        if with_c and not first:
            pltpu.make_async_copy(
                c_v[b], ci_out.at[pl.ds(ebase, _CHUNK)], sem_sc[b]).wait()
        pltpu.async_copy(ta.at[ii_v.at[lc]], a_v[b], sem_g[b])
        pltpu.async_copy(tb.at[jj_v.at[lc]], b_v[b], sem_g[b])
        if with_c:
            pltpu.async_copy(tc_.at[ii_v.at[lc]], c_v[b], sem_g[b])

    def finish(lc, b, first):
        off = ebase + lc * _CHUNK
        pltpu.make_async_copy(ta.at[ii_v.at[lc]], a_v[b], sem_g[b]).wait()
        pltpu.make_async_copy(tb.at[jj_v.at[lc]], b_v[b], sem_g[b]).wait()
        if with_c:
            pltpu.make_async_copy(tc_.at[ii_v.at[lc]], c_v[b],
                                  sem_g[b]).wait()
        if not first:
            pltpu.make_async_copy(
                o_v[b], g_out.at[pl.ds(ebase, _CHUNK)], sem_sg[b]).wait()

        def row(r, rc):
            for q in range(D // 16):
                sl = pl.ds(q * 16, 16)
                o_v[b][r, sl] = a_v[b][r, sl] + b_v[b][r, sl]
            return rc

        lax.fori_loop(0, _CHUNK, row, 0)
        pltpu.async_copy(o_v[b], g_out.at[pl.ds(off, _CHUNK)], sem_sg[b])
        if with_c:
            pltpu.async_copy(c_v[b], ci_out.at[pl.ds(off, _CHUNK)],
                             sem_sc[b])

    start(0, 0, True)
    start(1, 1, True)
    finish(0, 0, True)
    start(2, 0, False)
    finish(1, 1, True)
    start(3, 1, False)

    def body(t, carry):
        finish(2 * t + 2, 0, False)
        start(2 * t + 4, 0, False)
        finish(2 * t + 3, 1, False)
        start(2 * t + 5, 1, False)
        return carry

    lax.fori_loop(0, (_NCHUNK - 5) // 2, body, 0)
    finish(_NCHUNK - 3, 0, False)
    start(_NCHUNK - 1, 0, False)
    finish(_NCHUNK - 2, 1, False)
    finish(_NCHUNK - 1, 0, False)

    pltpu.make_async_copy(
        o_v[0], g_out.at[pl.ds(ebase, _CHUNK)], sem_sg[0]).wait()
    pltpu.make_async_copy(
        o_v[1], g_out.at[pl.ds(ebase, _CHUNK)], sem_sg[1]).wait()
    if with_c:
        pltpu.make_async_copy(
            c_v[0], ci_out.at[pl.ds(ebase, _CHUNK)], sem_sc[0]).wait()
        pltpu.make_async_copy(
            c_v[1], ci_out.at[pl.ds(ebase, _CHUNK)], sem_sc[1]).wait()


def _make_gather(with_c):
    out_type = [jax.ShapeDtypeStruct((N_EDGES, D), jnp.float32)]
    nbuf = 4 if with_c else 3
    scratch = [
        pltpu.VMEM((_NCHUNK, _CHUNK), jnp.int32),
        pltpu.VMEM((_NCHUNK, _CHUNK), jnp.int32),
    ] + [pltpu.VMEM((_CHUNK, D), jnp.float32)] * (2 * nbuf)
    if with_c:
        out_type = out_type + [jax.ShapeDtypeStruct((N_EDGES, D), jnp.float32)]
    scratch = scratch + [pltpu.SemaphoreType.DMA] * (6 if with_c else 4)
    return pl.kernel(
        functools.partial(_gather_body, with_c),
        mesh=_mesh,
        out_type=out_type,
        scratch_types=scratch,
    )


_gather_ac = _make_gather(True)
_gather_a = _make_gather(False)

_NPAD = 10240
_ROWS_PER_TILE = _NPAD // _NS
_ZROWS = 64


def _scatter_body(msg_hbm, ii2, out_hbm, acc_sh, ii_v,
                  m_v0, m_v1, z_v, sem_l0, sem_l1, sem_s0, sem_s1):
    cid = lax.axis_index("c")
    sid = lax.axis_index("s")
    m_v = (m_v0, m_v1)
    sem_l = (sem_l0, sem_l1)
    sem_s = (sem_s0, sem_s1)

    zv = jnp.zeros((16,), jnp.float32)

    def zrow(r, rc):
        for q in range(D // 16):
            z_v[r, pl.ds(q * 16, 16)] = zv
        return rc

    lax.fori_loop(0, _ZROWS, zrow, 0)
    for q in range(_ROWS_PER_TILE // _ZROWS):
        pltpu.sync_copy(
            z_v, acc_sh.at[pl.ds(sid * _ROWS_PER_TILE + q * _ZROWS, _ZROWS)])
    plsc.subcore_barrier()

    wid = cid * _NS + sid
    ebase = wid * _EPW
    pltpu.sync_copy(ii2.at[wid], ii_v)

    def start(lc, b, first):
        if not first:
            pltpu.make_async_copy(
                m_v[b], acc_sh.at[ii_v.at[lc]], sem_s[b]).wait()
        pltpu.async_copy(
            msg_hbm.at[pl.ds(ebase + lc * _CHUNK, _CHUNK)], m_v[b],
            sem_l[b])

    def finish(lc, b):
        pltpu.make_async_copy(
            msg_hbm.at[pl.ds(ebase, _CHUNK)], m_v[b], sem_l[b]).wait()
        pltpu.async_copy(m_v[b], acc_sh.at[ii_v.at[lc]], sem_s[b],
                         add=True)

    start(0, 0, True)
    start(1, 1, True)

    def body(t, carry):
        finish(2 * t, 0)
        start(2 * t + 2, 0, False)
        finish(2 * t + 1, 1)
        start(2 * t + 3, 1, False)
        return carry

    lax.fori_loop(0, (_NCHUNK - 3) // 2, body, 0)
    finish(_NCHUNK - 3, 0)
    start(_NCHUNK - 1, 0, False)
    finish(_NCHUNK - 2, 1)
    finish(_NCHUNK - 1, 0)
    pltpu.make_async_copy(m_v[0], acc_sh.at[ii_v.at[0]], sem_s[0]).wait()
    pltpu.make_async_copy(m_v[1], acc_sh.at[ii_v.at[0]], sem_s[1]).wait()
    plsc.subcore_barrier()

    for q in range(_ROWS_PER_TILE // _ZROWS):
        rows = sid * _ROWS_PER_TILE + q * _ZROWS
        pltpu.sync_copy(acc_sh.at[pl.ds(rows, _ZROWS)], z_v)
        pltpu.sync_copy(z_v, out_hbm.at[cid, pl.ds(rows, _ZROWS)])


_scatter_add = pl.kernel(
    _scatter_body,
    mesh=_mesh,
    out_type=jax.ShapeDtypeStruct((_NC, _NPAD, D), jnp.float32),
    scratch_types=[
        pltpu.VMEM_SHARED((_NPAD, D), jnp.float32),
        pltpu.VMEM((_NCHUNK, _CHUNK), jnp.int32),
        pltpu.VMEM((_CHUNK, D), jnp.float32),
        pltpu.VMEM((_CHUNK, D), jnp.float32),
        pltpu.VMEM((_ZROWS, D), jnp.float32),
        pltpu.SemaphoreType.DMA,
        pltpu.SemaphoreType.DMA,
        pltpu.SemaphoreType.DMA,
        pltpu.SemaphoreType.DMA,
    ],
)


def _np0_body(x_ref, wne, bne, wa, wb, wc, ta, tb, tc_):
    nf = jnp.maximum(
        jnp.dot(x_ref[...], wne[...], preferred_element_type=jnp.float32)
        + bne[...], 0.0)
    ta[...] = jnp.dot(nf, wa[...], preferred_element_type=jnp.float32)
    tb[...] = jnp.dot(nf, wb[...], preferred_element_type=jnp.float32)
    tc_[...] = jnp.dot(nf, wc[...], preferred_element_type=jnp.float32)


def _np1_body(p_ref, wa, wb, ta, tb):
    nf = p_ref[0] + p_ref[1]
    ta[...] = jnp.dot(nf, wa[...], preferred_element_type=jnp.float32)
    tb[...] = jnp.dot(nf, wb[...], preferred_element_type=jnp.float32)


_BE = 2560
_EGRID = N_EDGES // _BE


def _e0_body(g, ci, ea, wee, bee, we3, be_, wn2, bn_, wcls, bcls,
             t_out, msg_out, pred_out):
    ef0 = jnp.maximum(
        jnp.dot(ea[...], wee[...], preferred_element_type=jnp.float32)
        + bee[...], 0.0)
    t = BN_SCALE * jnp.maximum(
        g[...] + jnp.dot(ef0, we3[...], preferred_element_type=jnp.float32)
        + be_[...], 0.0)
    t_out[...] = t
    pred_out[...] = jnp.dot(t, wcls[...],
                            preferred_element_type=jnp.float32) + bcls[...]
    msg_out[...] = BN_SCALE * jnp.maximum(
        ci[...] + jnp.dot(t, wn2[...], preferred_element_type=jnp.float32)
        + bn_[...], 0.0)


def _e1_body(g, t0, we3, be_, wcls, bcls, pred_out):
    t = BN_SCALE * jnp.maximum(
        g[...] + jnp.dot(t0[...], we3[...], preferred_element_type=jnp.float32)
        + be_[...], 0.0)
    pred_out[...] = jnp.dot(t, wcls[...],
                            preferred_element_type=jnp.float32) + bcls[...]


def _full(shape):
    nd = len(shape)
    return pl.BlockSpec(shape, lambda i: (0,) * nd)


def _eblk(width):
    return pl.BlockSpec((_BE, width), lambda i: (i, 0))


def kernel(x, edge_attr, edge_index, W_ne, b_ne, W_ee, b_ee, W_e0, b_e0,
           W_n0, b_n0, W_e1, b_e1, W_n1, b_n1, W_cls, b_cls):
    f32 = jnp.float32
    jj2 = edge_index[0].astype(jnp.int32).reshape(_NW, _NCHUNK, _CHUNK)
    ii2 = edge_index[1].astype(jnp.int32).reshape(_NW, _NCHUNK, _CHUNK)

    wa0, wb0, we3_0 = W_e0[:D], W_e0[D:2 * D], W_e0[2 * D:]
    wc0, wn2_0 = W_n0[:D], W_n0[D:]
    wa1, wb1, we3_1 = W_e1[:D], W_e1[D:2 * D], W_e1[2 * D:]
    b_ne2 = b_ne.reshape(1, D)
    b_ee2 = b_ee.reshape(1, D)
    b_e02 = b_e0.reshape(1, D)
    b_n02 = b_n0.reshape(1, D)
    b_e12 = b_e1.reshape(1, D)
    b_cls2 = b_cls.reshape(1, 1)

    ta0, tb0, tc0 = pl.pallas_call(
        _np0_body,
        out_shape=[jax.ShapeDtypeStruct((N_NODES, D), f32)] * 3,
    )(x, W_ne, b_ne2, wa0, wb0, wc0)

    g0, ci0 = _gather_ac(ta0, tb0, tc0, ii2, jj2)

    t0, msg, pred0 = pl.pallas_call(
        _e0_body,
        grid=(_EGRID,),
        in_specs=[
            _eblk(D), _eblk(D), _eblk(16),
            _full((16, D)), _full((1, D)), _full((D, D)), _full((1, D)),
            _full((D, D)), _full((1, D)), _full((D, 1)), _full((1, 1)),
        ],
        out_specs=[_eblk(D), _eblk(D), _eblk(1)],
        out_shape=[
            jax.ShapeDtypeStruct((N_EDGES, D), f32),
            jax.ShapeDtypeStruct((N_EDGES, D), f32),
            jax.ShapeDtypeStruct((N_EDGES, 1), f32),
        ],
    )(g0, ci0, edge_attr, W_ee, b_ee2, we3_0, b_e02, wn2_0, b_n02,
      W_cls, b_cls2)

    partials = _scatter_add(msg, ii2)[:, :N_NODES]

    ta1, tb1 = pl.pallas_call(
        _np1_body,
        out_shape=[jax.ShapeDtypeStruct((N_NODES, D), f32)] * 2,
    )(partials, wa1, wb1)

    (g1,) = _gather_a(ta1, tb1, ii2, jj2)

    pred1 = pl.pallas_call(
        _e1_body,
        grid=(_EGRID,),
        in_specs=[
            _eblk(D), _eblk(D),
            _full((D, D)), _full((1, D)), _full((D, 1)), _full((1, 1)),
        ],
        out_specs=_eblk(1),
        out_shape=jax.ShapeDtypeStruct((N_EDGES, 1), f32),
    )(g1, t0, we3_1, b_e12, W_cls, b_cls2)

    return (pred0.reshape(N_EDGES), pred1.reshape(N_EDGES))
